# baseline (speedup 1.0000x reference)
import functools

import jax
import jax.numpy as jnp
from jax import lax
from jax.experimental import pallas as pl
from jax.experimental.pallas import tpu as pltpu
from jax.experimental.pallas import tpu_sc as plsc

N = 10000
D = 128
H = 8
C = 16
E = 320000
ROW = 144
NC = 2
NS = 16
NW = NC * NS
CH = 64
CPW = 162
E_PAD = NW * CPW * CH
N_TAB = 10016
N_ACC = 10240
RPS = N_ACC // NS


def _prep_body(x_ref, w_ref, a2_ref, r_ref, hs_ref, adst_ref, self_ref):
    h = jnp.dot(x_ref[...], w_ref[...], preferred_element_type=jnp.float32)
    a2 = jnp.dot(h, a2_ref[...], preferred_element_type=jnp.float32)
    asrc = a2[:, 0:16]
    adst = a2[:, 16:32]
    hs_ref[:, 0:D] = h
    hs_ref[:, D:ROW] = asrc
    adst_ref[...] = adst
    al = asrc + adst
    al = jnp.maximum(al, 0.2 * al)
    lane = lax.broadcasted_iota(jnp.int32, (1, 16), 1)
    wself = jnp.exp(al) * (lane < H).astype(jnp.float32)
    self_ref[:, 0:D] = h * jnp.dot(wself, r_ref[...],
                                   preferred_element_type=jnp.float32)
    self_ref[:, D:ROW] = wself


def _prep_call(xp, W, A2, R):
    bp = N_TAB // 4
    return pl.pallas_call(
        _prep_body,
        grid=(N_TAB // bp,),
        in_specs=[
            pl.BlockSpec((bp, D), lambda i: (i, 0)),
            pl.BlockSpec((D, D), lambda i: (0, 0)),
            pl.BlockSpec((D, 32), lambda i: (0, 0)),
            pl.BlockSpec((16, D), lambda i: (0, 0)),
        ],
        out_specs=[
            pl.BlockSpec((bp, ROW), lambda i: (i, 0)),
            pl.BlockSpec((bp, 16), lambda i: (i, 0)),
            pl.BlockSpec((bp, ROW), lambda i: (i, 0)),
        ],
        out_shape=[
            jax.ShapeDtypeStruct((N_TAB, ROW), jnp.float32),
            jax.ShapeDtypeStruct((N_TAB, 16), jnp.float32),
            jax.ShapeDtypeStruct((N_TAB, ROW), jnp.float32),
        ],
    )(xp, W, A2, R)


NBUF = 3
GROUPS = CPW // NBUF


def _edge_body(src_ref, dst_ref, hs_ref, adst_ref, out_ref,
               g0, g1, g2, d0, d1, d2, si0, si1, si2, di0, di1, di2,
               sc0, sc1, sc2, acc,
               sg0, sg1, sg2, ss0, ss1, ss2, sei0, sei1, sei2):
    gbufs = (g0, g1, g2)
    dbufs = (d0, d1, d2)
    sidxs = (si0, si1, si2)
    didxs = (di0, di1, di2)
    dscats = (sc0, sc1, sc2)
    sgs = (sg0, sg1, sg2)
    sss = (ss0, ss1, ss2)
    seis = (sei0, sei1, sei2)
    c = lax.axis_index("c")
    s = lax.axis_index("s")
    wid = s * NC + c
    groups = ROW // 16

    def zgb(i, carry):
        g0[i // groups, pl.ds((i % groups) * 16, 16)] = jnp.zeros((16,), jnp.float32)
        return carry
    lax.fori_loop(0, CH * groups, zgb, 0)
    r0 = s * RPS
    for t in range(RPS // CH):
        pltpu.sync_copy(g0, acc.at[pl.ds(r0 + t * CH, CH)])
    plsc.subcore_barrier()

    base = wid * (CPW * CH)

    def start_idx(ci, t):
        off = base + ci * CH
        pltpu.async_copy(src_ref.at[pl.ds(off, CH)], sidxs[t], seis[t])
        pltpu.async_copy(dst_ref.at[pl.ds(off, CH)], didxs[t], seis[t])

    def wait_idx(t):
        pltpu.make_async_copy(src_ref.at[pl.ds(0, CH)], sidxs[t], seis[t]).wait()
        pltpu.make_async_copy(dst_ref.at[pl.ds(0, CH)], didxs[t], seis[t]).wait()

    def start_gather(t):
        pltpu.async_copy(hs_ref.at[sidxs[t]], gbufs[t], sgs[t])
        pltpu.async_copy(adst_ref.at[didxs[t]], dbufs[t], sgs[t])

    def wait_gather(t):
        pltpu.make_async_copy(hs_ref.at[pl.ds(0, CH)], gbufs[t], sgs[t]).wait()
        pltpu.make_async_copy(adst_ref.at[pl.ds(0, CH)], dbufs[t], sgs[t]).wait()

    def wait_scatter(t):
        pltpu.make_async_copy(hs_ref.at[pl.ds(0, CH)], gbufs[t], sss[t]).wait()

    for t in range(NBUF):
        start_idx(jnp.int32(t), t)
    for t in range(NBUF - 1):
        wait_idx(t)
        start_gather(t)

    def group(g, carry):
        for t in range(NBUF):
            ci = NBUF * g + t
            gbuf = gbufs[t]
            dbuf = dbufs[t]
            didx = didxs[t]
            dscat = dscats[t]
            fb = (t + NBUF - 1) % NBUF
            wait_gather(t)
            for i in range(CH // 16):
                dscat[pl.ds(i * 16, 16)] = didx[pl.ds(i * 16, 16)]

            @plsc.parallel_loop(0, CH, 1, unroll=8)
            def edge(k):
                al = gbuf[k, pl.ds(D, 16)] + dbuf[k, :]
                al = jnp.maximum(al, 0.2 * al)
                w = jnp.exp(al)
                gbuf[k, pl.ds(D, 16)] = w
                dn = lax.GatherDimensionNumbers(
                    offset_dims=(), collapsed_slice_dims=(0,),
                    start_index_map=(0,))
                for j in range(H):
                    wj = lax.gather(
                        w, jnp.full((16, 1), j, jnp.int32), dn, slice_sizes=(1,),
                        mode=lax.GatherScatterMode.PROMISE_IN_BOUNDS)
                    gbuf[k, pl.ds(j * 16, 16)] = gbuf[k, pl.ds(j * 16, 16)] * wj
            pltpu.async_copy(gbuf, acc.at[dscat], sss[t], add=True)

            if t == 0:
                @pl.when(g > 0)
                def _():
                    wait_scatter(fb)
                wait_idx(fb)
                start_gather(fb)
            else:
                wait_scatter(fb)

                @pl.when(g < GROUPS - 1)
                def _():
                    wait_idx(fb)
                    start_gather(fb)

            @pl.when(g < GROUPS - 1)
            def _():
                start_idx(ci + NBUF, t)
        return carry
    lax.fori_loop(0, GROUPS, group, 0)

    wait_scatter(NBUF - 1)
    plsc.subcore_barrier()
    o0 = c * N_ACC + r0
    for t in range(RPS // CH):
        pltpu.sync_copy(acc.at[pl.ds(r0 + t * CH, CH)],
                        out_ref.at[pl.ds(o0 + t * CH, CH)])


def _edge_call(srcp, dstp, hs, adst):
    mesh = plsc.VectorSubcoreMesh(core_axis_name="c", subcore_axis_name="s",
                                  num_cores=NC, num_subcores=NS)
    f = pl.kernel(
        _edge_body,
        out_type=jax.ShapeDtypeStruct((NC * N_ACC, ROW), jnp.float32),
        mesh=mesh,
        compiler_params=pltpu.CompilerParams(use_tc_tiling_on_sc=False,
                                             needs_layout_passes=False),
        scratch_types=(
            [pltpu.VMEM((CH, ROW), jnp.float32) for _ in range(NBUF)]
            + [pltpu.VMEM((CH, 16), jnp.float32) for _ in range(NBUF)]
            + [pltpu.VMEM((CH,), jnp.int32) for _ in range(3 * NBUF)]
            + [pltpu.VMEM_SHARED((N_ACC, ROW), jnp.float32)]
            + [pltpu.SemaphoreType.DMA for _ in range(3 * NBUF)]
        ),
    )
    return f(srcp, dstp, hs, adst)


def _norm1_body(a0_ref, a1_ref, sf_ref, b_ref, r_ref, o_ref, st_ref):
    tot = a0_ref[...] + a1_ref[...] + sf_ref[...]
    den = jnp.dot(tot[:, D:ROW], r_ref[...], preferred_element_type=jnp.float32)
    o = tot[:, 0:D] / den + b_ref[...]
    o_ref[...] = o
    st_ref[0, 0:1, :] = jnp.sum(o, axis=0, keepdims=True)
    st_ref[0, 1:2, :] = jnp.sum(o * o, axis=0, keepdims=True)


def _norm1_call(a0, a1, sf, bias1, R):
    bp = 2000
    return pl.pallas_call(
        _norm1_body,
        grid=(N // bp,),
        in_specs=[
            pl.BlockSpec((bp, ROW), lambda i: (i, 0)),
            pl.BlockSpec((bp, ROW), lambda i: (i, 0)),
            pl.BlockSpec((bp, ROW), lambda i: (i, 0)),
            pl.BlockSpec((1, D), lambda i: (0, 0)),
            pl.BlockSpec((16, D), lambda i: (0, 0)),
        ],
        out_specs=[
            pl.BlockSpec((bp, D), lambda i: (i, 0)),
            pl.BlockSpec((1, 2, D), lambda i: (i, 0, 0)),
        ],
        out_shape=[
            jax.ShapeDtypeStruct((N, D), jnp.float32),
            jax.ShapeDtypeStruct((N // bp, 2, D), jnp.float32),
        ],
    )(a0, a1, sf, bias1, R)


def _norm2_body(o_ref, st_ref, x_ref, g_ref, be_ref, out_ref):
    st = jnp.sum(st_ref[...], axis=0)
    m = st[0:1, :] * (1.0 / N)
    v = st[1:2, :] * (1.0 / N) - m * m
    inv = lax.rsqrt(v + 1e-5)
    out_ref[...] = (o_ref[...] - m) * inv * g_ref[...] + be_ref[...] + x_ref[...]


def _norm2_call(o, st, x, g1, b1):
    bp = 2000
    nb = N // bp
    return pl.pallas_call(
        _norm2_body,
        grid=(nb,),
        in_specs=[
            pl.BlockSpec((bp, D), lambda i: (i, 0)),
            pl.BlockSpec((nb, 2, D), lambda i: (0, 0, 0)),
            pl.BlockSpec((bp, D), lambda i: (i, 0)),
            pl.BlockSpec((1, D), lambda i: (0, 0)),
            pl.BlockSpec((1, D), lambda i: (0, 0)),
        ],
        out_specs=pl.BlockSpec((bp, D), lambda i: (i, 0)),
        out_shape=jax.ShapeDtypeStruct((N, D), jnp.float32),
    )(o, st, x, g1, b1)


def kernel(x, edge_index, W, att_src, att_dst, bias, gamma, beta):
    f32 = jnp.float32
    xp = jnp.zeros((N_TAB, D), f32).at[:N].set(x)
    oh16 = (jnp.arange(D)[:, None] // C == jnp.arange(16)[None, :]).astype(f32)
    A2 = jnp.concatenate([att_src.reshape(D)[:, None] * oh16,
                          att_dst.reshape(D)[:, None] * oh16], axis=1)
    R = oh16.T
    hs, adst, selfacc = _prep_call(xp, W, A2, R)

    pad = E_PAD - E
    padidx = (jnp.arange(pad, dtype=jnp.int32) % 8) + N
    srcp = jnp.concatenate([edge_index[0], padidx])
    dstp = jnp.concatenate([edge_index[1], padidx])
    accs = _edge_call(srcp, dstp, hs, adst)

    o, st = _norm1_call(accs[:N], accs[N_ACC:N_ACC + N], selfacc[:N],
                        bias.reshape(1, D), R)
    return _norm2_call(o, st, x, gamma.reshape(1, D), beta.reshape(1, D))


# speedup vs baseline: 1.2224x; 1.2224x over previous
import functools

import jax
import jax.numpy as jnp
from jax import lax
from jax.experimental import pallas as pl
from jax.experimental.pallas import tpu as pltpu
from jax.experimental.pallas import tpu_sc as plsc

N = 10000
D = 128
H = 8
C = 16
E = 320000
ROW = 144
NC = 2
NS = 16
NW = NC * NS
CH = 64
CPW = 162
E_PAD = NW * CPW * CH
N_TAB = 10016
N_ACC = 10240
RPS = N_ACC // NS


def _prep_body(x_ref, w_ref, a2_ref, r_ref, hs_ref, adst_ref, self_ref):
    h = jnp.dot(x_ref[...], w_ref[...], preferred_element_type=jnp.float32)
    a2 = jnp.dot(h, a2_ref[...], preferred_element_type=jnp.float32)
    asrc = a2[:, 0:16]
    adst = a2[:, 16:32]
    hs_ref[:, 0:D] = h
    hs_ref[:, D:ROW] = asrc
    adst_ref[...] = adst
    al = asrc + adst
    al = jnp.maximum(al, 0.2 * al)
    lane = lax.broadcasted_iota(jnp.int32, (1, 16), 1)
    wself = jnp.exp(al) * (lane < H).astype(jnp.float32)
    self_ref[:, 0:D] = h * jnp.dot(wself, r_ref[...],
                                   preferred_element_type=jnp.float32)
    self_ref[:, D:ROW] = wself


def _prep_call(xp, W, A2, R):
    bp = N_TAB // 4
    return pl.pallas_call(
        _prep_body,
        grid=(N_TAB // bp,),
        in_specs=[
            pl.BlockSpec((bp, D), lambda i: (i, 0)),
            pl.BlockSpec((D, D), lambda i: (0, 0)),
            pl.BlockSpec((D, 32), lambda i: (0, 0)),
            pl.BlockSpec((16, D), lambda i: (0, 0)),
        ],
        out_specs=[
            pl.BlockSpec((bp, ROW), lambda i: (i, 0)),
            pl.BlockSpec((bp, 16), lambda i: (i, 0)),
            pl.BlockSpec((bp, ROW), lambda i: (i, 0)),
        ],
        out_shape=[
            jax.ShapeDtypeStruct((N_TAB, ROW), jnp.float32),
            jax.ShapeDtypeStruct((N_TAB, 16), jnp.float32),
            jax.ShapeDtypeStruct((N_TAB, ROW), jnp.float32),
        ],
    )(xp, W, A2, R)


NBUF = 3
GROUPS = CPW // NBUF


def _edge_body(src_ref, dst_ref, hs_ref, adst_ref, out0_ref, out1_ref,
               g0, g1, g2, d0, d1, d2, si0, si1, si2, di0, di1, di2,
               sc0, sc1, sc2, acc,
               sg0, sg1, sg2, ss0, ss1, ss2, sei0, sei1, sei2):
    gbufs = (g0, g1, g2)
    dbufs = (d0, d1, d2)
    sidxs = (si0, si1, si2)
    didxs = (di0, di1, di2)
    dscats = (sc0, sc1, sc2)
    sgs = (sg0, sg1, sg2)
    sss = (ss0, ss1, ss2)
    seis = (sei0, sei1, sei2)
    c = lax.axis_index("c")
    s = lax.axis_index("s")
    wid = s * NC + c
    groups = ROW // 16

    def zgb(i, carry):
        g0[i // groups, pl.ds((i % groups) * 16, 16)] = jnp.zeros((16,), jnp.float32)
        return carry
    lax.fori_loop(0, CH * groups, zgb, 0)
    r0 = s * RPS
    for t in range(RPS // CH):
        pltpu.sync_copy(g0, acc.at[pl.ds(r0 + t * CH, CH)])
    plsc.subcore_barrier()

    base = wid * (CPW * CH)

    def start_idx(ci, t):
        off = base + ci * CH
        pltpu.async_copy(src_ref.at[pl.ds(off, CH)], sidxs[t], seis[t])
        pltpu.async_copy(dst_ref.at[pl.ds(off, CH)], didxs[t], seis[t])

    def wait_idx(t):
        pltpu.make_async_copy(src_ref.at[pl.ds(0, CH)], sidxs[t], seis[t]).wait()
        pltpu.make_async_copy(dst_ref.at[pl.ds(0, CH)], didxs[t], seis[t]).wait()

    def start_gather(t):
        pltpu.async_copy(hs_ref.at[sidxs[t]], gbufs[t], sgs[t])
        pltpu.async_copy(adst_ref.at[didxs[t]], dbufs[t], sgs[t])

    def wait_gather(t):
        pltpu.make_async_copy(hs_ref.at[pl.ds(0, CH)], gbufs[t], sgs[t]).wait()
        pltpu.make_async_copy(adst_ref.at[pl.ds(0, CH)], dbufs[t], sgs[t]).wait()

    def wait_scatter(t):
        pltpu.make_async_copy(hs_ref.at[pl.ds(0, CH)], gbufs[t], sss[t]).wait()

    for t in range(NBUF):
        start_idx(jnp.int32(t), t)
    for t in range(NBUF - 1):
        wait_idx(t)
        start_gather(t)

    def group(g, carry):
        for t in range(NBUF):
            ci = NBUF * g + t
            gbuf = gbufs[t]
            dbuf = dbufs[t]
            didx = didxs[t]
            dscat = dscats[t]
            fb = (t + NBUF - 1) % NBUF
            wait_gather(t)
            for i in range(CH // 16):
                dscat[pl.ds(i * 16, 16)] = didx[pl.ds(i * 16, 16)]

            @plsc.parallel_loop(0, CH, 1, unroll=4)
            def edge(k):
                al = gbuf[k, pl.ds(D, 16)] + dbuf[k, :]
                al = jnp.maximum(al, 0.2 * al)
                w = jnp.exp(al)
                gbuf[k, pl.ds(D, 16)] = w
                dn = lax.GatherDimensionNumbers(
                    offset_dims=(), collapsed_slice_dims=(0,),
                    start_index_map=(0,))
                for j in range(H):
                    wj = lax.gather(
                        w, jnp.full((16, 1), j, jnp.int32), dn, slice_sizes=(1,),
                        mode=lax.GatherScatterMode.PROMISE_IN_BOUNDS)
                    gbuf[k, pl.ds(j * 16, 16)] = gbuf[k, pl.ds(j * 16, 16)] * wj
            pltpu.async_copy(gbuf, acc.at[dscat], sss[t], add=True)

            if t == 0:
                @pl.when(g > 0)
                def _():
                    wait_scatter(fb)
                wait_idx(fb)
                start_gather(fb)
            else:
                wait_scatter(fb)

                @pl.when(g < GROUPS - 1)
                def _():
                    wait_idx(fb)
                    start_gather(fb)

            @pl.when(g < GROUPS - 1)
            def _():
                start_idx(ci + NBUF, t)
        return carry
    lax.fori_loop(0, GROUPS, group, 0)

    wait_scatter(NBUF - 1)
    plsc.subcore_barrier()

    @pl.when(c == 0)
    def _():
        for t in range(RPS // CH):
            pltpu.sync_copy(acc.at[pl.ds(r0 + t * CH, CH)],
                            out0_ref.at[pl.ds(r0 + t * CH, CH)])

    @pl.when(c == 1)
    def _():
        for t in range(RPS // CH):
            pltpu.sync_copy(acc.at[pl.ds(r0 + t * CH, CH)],
                            out1_ref.at[pl.ds(r0 + t * CH, CH)])


def _edge_call(srcp, dstp, hs, adst):
    mesh = plsc.VectorSubcoreMesh(core_axis_name="c", subcore_axis_name="s",
                                  num_cores=NC, num_subcores=NS)
    f = pl.kernel(
        _edge_body,
        out_type=[jax.ShapeDtypeStruct((N_ACC, ROW), jnp.float32),
                  jax.ShapeDtypeStruct((N_ACC, ROW), jnp.float32)],
        mesh=mesh,
        compiler_params=pltpu.CompilerParams(use_tc_tiling_on_sc=False,
                                             needs_layout_passes=False),
        scratch_types=(
            [pltpu.VMEM((CH, ROW), jnp.float32) for _ in range(NBUF)]
            + [pltpu.VMEM((CH, 16), jnp.float32) for _ in range(NBUF)]
            + [pltpu.VMEM((CH,), jnp.int32) for _ in range(3 * NBUF)]
            + [pltpu.VMEM_SHARED((N_ACC, ROW), jnp.float32)]
            + [pltpu.SemaphoreType.DMA for _ in range(3 * NBUF)]
        ),
    )
    return f(srcp, dstp, hs, adst)


def _norm1_body(a0_ref, a1_ref, sf_ref, b_ref, r_ref, o_ref, st_ref):
    tot = a0_ref[...] + a1_ref[...] + sf_ref[...]
    den = jnp.dot(tot[:, D:ROW], r_ref[...], preferred_element_type=jnp.float32)
    o = tot[:, 0:D] / den + b_ref[...]
    o_ref[...] = o
    st_ref[0, 0:1, :] = jnp.sum(o, axis=0, keepdims=True)
    st_ref[0, 1:2, :] = jnp.sum(o * o, axis=0, keepdims=True)


def _norm1_call(a0, a1, sf, bias1, R):
    bp = 2000
    return pl.pallas_call(
        _norm1_body,
        grid=(N // bp,),
        in_specs=[
            pl.BlockSpec((bp, ROW), lambda i: (i, 0)),
            pl.BlockSpec((bp, ROW), lambda i: (i, 0)),
            pl.BlockSpec((bp, ROW), lambda i: (i, 0)),
            pl.BlockSpec((1, D), lambda i: (0, 0)),
            pl.BlockSpec((16, D), lambda i: (0, 0)),
        ],
        out_specs=[
            pl.BlockSpec((bp, D), lambda i: (i, 0)),
            pl.BlockSpec((1, 2, D), lambda i: (i, 0, 0)),
        ],
        out_shape=[
            jax.ShapeDtypeStruct((N, D), jnp.float32),
            jax.ShapeDtypeStruct((N // bp, 2, D), jnp.float32),
        ],
    )(a0, a1, sf, bias1, R)


def _norm2_body(o_ref, st_ref, x_ref, g_ref, be_ref, out_ref):
    st = jnp.sum(st_ref[...], axis=0)
    m = st[0:1, :] * (1.0 / N)
    v = st[1:2, :] * (1.0 / N) - m * m
    inv = lax.rsqrt(v + 1e-5)
    out_ref[...] = (o_ref[...] - m) * inv * g_ref[...] + be_ref[...] + x_ref[...]


def _norm2_call(o, st, x, g1, b1):
    bp = 2000
    nb = N // bp
    return pl.pallas_call(
        _norm2_body,
        grid=(nb,),
        in_specs=[
            pl.BlockSpec((bp, D), lambda i: (i, 0)),
            pl.BlockSpec((nb, 2, D), lambda i: (0, 0, 0)),
            pl.BlockSpec((bp, D), lambda i: (i, 0)),
            pl.BlockSpec((1, D), lambda i: (0, 0)),
            pl.BlockSpec((1, D), lambda i: (0, 0)),
        ],
        out_specs=pl.BlockSpec((bp, D), lambda i: (i, 0)),
        out_shape=jax.ShapeDtypeStruct((N, D), jnp.float32),
    )(o, st, x, g1, b1)


def kernel(x, edge_index, W, att_src, att_dst, bias, gamma, beta):
    f32 = jnp.float32
    xp = jnp.zeros((N_TAB, D), f32).at[:N].set(x)
    oh16 = (jnp.arange(D)[:, None] // C == jnp.arange(16)[None, :]).astype(f32)
    A2 = jnp.concatenate([att_src.reshape(D)[:, None] * oh16,
                          att_dst.reshape(D)[:, None] * oh16], axis=1)
    R = oh16.T
    hs, adst, selfacc = _prep_call(xp, W, A2, R)

    pad = E_PAD - E
    padidx = (jnp.arange(pad, dtype=jnp.int32) % 8) + N
    srcp = jnp.concatenate([edge_index[0], padidx])
    dstp = jnp.concatenate([edge_index[1], padidx])
    acc0, acc1 = _edge_call(srcp, dstp, hs, adst)

    o, st = _norm1_call(acc0, acc1, selfacc, bias.reshape(1, D), R)
    return _norm2_call(o, st, x, gamma.reshape(1, D), beta.reshape(1, D))


# speedup vs baseline: 1.3095x; 1.0713x over previous
import functools

import jax
import jax.numpy as jnp
from jax import lax
from jax.experimental import pallas as pl
from jax.experimental.pallas import tpu as pltpu
from jax.experimental.pallas import tpu_sc as plsc

N = 10000
D = 128
H = 8
C = 16
E = 320000
ROW = 144
NC = 2
NS = 16
NW = NC * NS
CH = 64
CPW = 162
E_PAD = NW * CPW * CH
N_TAB = 10016
N_ACC = 10240
RPS = N_ACC // NS


def _prep_body(x_ref, w_ref, a2_ref, r_ref, hs_ref, adst_ref, self_ref):
    h = jnp.dot(x_ref[...], w_ref[...], preferred_element_type=jnp.float32)
    a2 = jnp.dot(h, a2_ref[...], preferred_element_type=jnp.float32)
    asrc = a2[:, 0:16]
    adst = a2[:, 16:32]
    hs_ref[:, 0:D] = h
    hs_ref[:, D:ROW] = asrc
    adst_ref[...] = adst
    al = asrc + adst
    al = jnp.maximum(al, 0.2 * al)
    lane = lax.broadcasted_iota(jnp.int32, (1, 16), 1)
    wself = jnp.exp(al) * (lane < H).astype(jnp.float32)
    self_ref[:, 0:D] = h * jnp.dot(wself, r_ref[...],
                                   preferred_element_type=jnp.float32)
    self_ref[:, D:ROW] = wself


def _prep_call(xp, W, A2, R):
    bp = N_TAB // 4
    return pl.pallas_call(
        _prep_body,
        grid=(N_TAB // bp,),
        in_specs=[
            pl.BlockSpec((bp, D), lambda i: (i, 0)),
            pl.BlockSpec((D, D), lambda i: (0, 0)),
            pl.BlockSpec((D, 32), lambda i: (0, 0)),
            pl.BlockSpec((16, D), lambda i: (0, 0)),
        ],
        out_specs=[
            pl.BlockSpec((bp, ROW), lambda i: (i, 0)),
            pl.BlockSpec((bp, 16), lambda i: (i, 0)),
            pl.BlockSpec((bp, ROW), lambda i: (i, 0)),
        ],
        out_shape=[
            jax.ShapeDtypeStruct((N_TAB, ROW), jnp.float32),
            jax.ShapeDtypeStruct((N_TAB, 16), jnp.float32),
            jax.ShapeDtypeStruct((N_TAB, ROW), jnp.float32),
        ],
    )(xp, W, A2, R)


NBUF = 3
GROUPS = CPW // NBUF


def _edge_body(ei_ref, hs_ref, adst_ref,
               outa0_ref, outb0_ref, outa1_ref, outb1_ref,
               g0, g1, g2, d0, d1, d2, si0, si1, si2, di0, di1, di2,
               sc0, sc1, sc2, acc,
               sg0, sg1, sg2, ss0, ss1, ss2, sei0, sei1, sei2):
    gbufs = (g0, g1, g2)
    dbufs = (d0, d1, d2)
    sidxs = (si0, si1, si2)
    didxs = (di0, di1, di2)
    dscats = (sc0, sc1, sc2)
    sgs = (sg0, sg1, sg2)
    sss = (ss0, ss1, ss2)
    seis = (sei0, sei1, sei2)
    c = lax.axis_index("c")
    s = lax.axis_index("s")
    wid = s * NC + c
    groups = ROW // 16

    def zgb(i, carry):
        g0[i // groups, pl.ds((i % groups) * 16, 16)] = jnp.zeros((16,), jnp.float32)
        return carry
    lax.fori_loop(0, CH * groups, zgb, 0)
    r0 = s * RPS
    for t in range(RPS // CH):
        pltpu.sync_copy(g0, acc.at[pl.ds(r0 + t * CH, CH)])
    plsc.subcore_barrier()

    base = wid * (CPW * CH)

    def start_idx(ci, t):
        off = base + ci * CH

        @pl.when(off < E)
        def _():
            pltpu.async_copy(ei_ref.at[pl.ds(off, CH)], sidxs[t], seis[t])
            pltpu.async_copy(ei_ref.at[pl.ds(E + off, CH)], didxs[t], seis[t])

        @pl.when(off >= E)
        def _():
            for i in range(CH // 16):
                padv = N + (lax.iota(jnp.int32, 16) & 7)
                sidxs[t][pl.ds(i * 16, 16)] = padv
                didxs[t][pl.ds(i * 16, 16)] = padv

    def wait_idx(ci, t):
        off = base + ci * CH

        @pl.when(off < E)
        def _():
            pltpu.make_async_copy(ei_ref.at[pl.ds(0, CH)], sidxs[t], seis[t]).wait()
            pltpu.make_async_copy(ei_ref.at[pl.ds(0, CH)], didxs[t], seis[t]).wait()

    def start_gather(t):
        pltpu.async_copy(hs_ref.at[sidxs[t]], gbufs[t], sgs[t])
        pltpu.async_copy(adst_ref.at[didxs[t]], dbufs[t], sgs[t])

    def wait_gather(t):
        pltpu.make_async_copy(hs_ref.at[pl.ds(0, CH)], gbufs[t], sgs[t]).wait()
        pltpu.make_async_copy(adst_ref.at[pl.ds(0, CH)], dbufs[t], sgs[t]).wait()

    def wait_scatter(t):
        pltpu.make_async_copy(hs_ref.at[pl.ds(0, CH)], gbufs[t], sss[t]).wait()

    for t in range(NBUF):
        start_idx(jnp.int32(t), t)
    for t in range(NBUF - 1):
        wait_idx(jnp.int32(t), t)
        start_gather(t)

    def group(g, carry):
        for t in range(NBUF):
            ci = NBUF * g + t
            gbuf = gbufs[t]
            dbuf = dbufs[t]
            didx = didxs[t]
            dscat = dscats[t]
            fb = (t + NBUF - 1) % NBUF
            wait_gather(t)
            for i in range(CH // 16):
                dscat[pl.ds(i * 16, 16)] = didx[pl.ds(i * 16, 16)]

            @plsc.parallel_loop(0, CH, 1, unroll=4)
            def edge(k):
                al = gbuf[k, pl.ds(D, 16)] + dbuf[k, :]
                al = jnp.maximum(al, 0.2 * al)
                w = jnp.exp(al)
                gbuf[k, pl.ds(D, 16)] = w
                dn = lax.GatherDimensionNumbers(
                    offset_dims=(), collapsed_slice_dims=(0,),
                    start_index_map=(0,))
                for j in range(H):
                    wj = lax.gather(
                        w, jnp.full((16, 1), j, jnp.int32), dn, slice_sizes=(1,),
                        mode=lax.GatherScatterMode.PROMISE_IN_BOUNDS)
                    gbuf[k, pl.ds(j * 16, 16)] = gbuf[k, pl.ds(j * 16, 16)] * wj
            pltpu.async_copy(gbuf, acc.at[dscat], sss[t], add=True)

            if t == 0:
                @pl.when(g > 0)
                def _():
                    wait_scatter(fb)
                wait_idx(ci + NBUF - 1, fb)
                start_gather(fb)
            else:
                wait_scatter(fb)

                @pl.when(g < GROUPS - 1)
                def _():
                    wait_idx(ci + NBUF - 1, fb)
                    start_gather(fb)

            @pl.when(g < GROUPS - 1)
            def _():
                start_idx(ci + NBUF, t)
        return carry
    lax.fori_loop(0, GROUPS, group, 0)

    wait_scatter(NBUF - 1)
    plsc.subcore_barrier()

    @pl.when(c == 0)
    def _():
        for t in range(RPS // CH):
            rr = r0 + t * CH
            pltpu.sync_copy(acc.at[pl.ds(rr, CH), pl.ds(0, D)],
                            outa0_ref.at[pl.ds(rr, CH)])
            pltpu.sync_copy(acc.at[pl.ds(rr, CH), pl.ds(D, 16)],
                            outb0_ref.at[pl.ds(rr, CH)])

    @pl.when(c == 1)
    def _():
        for t in range(RPS // CH):
            rr = r0 + t * CH
            pltpu.sync_copy(acc.at[pl.ds(rr, CH), pl.ds(0, D)],
                            outa1_ref.at[pl.ds(rr, CH)])
            pltpu.sync_copy(acc.at[pl.ds(rr, CH), pl.ds(D, 16)],
                            outb1_ref.at[pl.ds(rr, CH)])


def _edge_call(eiflat, hs, adst):
    mesh = plsc.VectorSubcoreMesh(core_axis_name="c", subcore_axis_name="s",
                                  num_cores=NC, num_subcores=NS)
    f = pl.kernel(
        _edge_body,
        out_type=[jax.ShapeDtypeStruct((N_ACC, D), jnp.float32),
                  jax.ShapeDtypeStruct((N_ACC, 16), jnp.float32),
                  jax.ShapeDtypeStruct((N_ACC, D), jnp.float32),
                  jax.ShapeDtypeStruct((N_ACC, 16), jnp.float32)],
        mesh=mesh,
        compiler_params=pltpu.CompilerParams(use_tc_tiling_on_sc=False,
                                             needs_layout_passes=False),
        scratch_types=(
            [pltpu.VMEM((CH, ROW), jnp.float32) for _ in range(NBUF)]
            + [pltpu.VMEM((CH, 16), jnp.float32) for _ in range(NBUF)]
            + [pltpu.VMEM((CH,), jnp.int32) for _ in range(3 * NBUF)]
            + [pltpu.VMEM_SHARED((N_ACC, ROW), jnp.float32)]
            + [pltpu.SemaphoreType.DMA for _ in range(3 * NBUF)]
        ),
    )
    return f(eiflat, hs, adst)


def _norm1_body(a0_ref, a1_ref, b0_ref, b1_ref, sf_ref, b_ref, r_ref,
                o_ref, st_ref):
    tot = a0_ref[...] + a1_ref[...] + sf_ref[:, 0:D]
    den16 = b0_ref[...] + b1_ref[...] + sf_ref[:, D:ROW]
    den = jnp.dot(den16, r_ref[...], preferred_element_type=jnp.float32)
    o = tot / den + b_ref[...]
    o_ref[...] = o
    st_ref[0, 0:1, :] = jnp.sum(o, axis=0, keepdims=True)
    st_ref[0, 1:2, :] = jnp.sum(o * o, axis=0, keepdims=True)


def _norm1_call(a0, a1, b0, b1, sf, bias1, R):
    bp = 2000
    return pl.pallas_call(
        _norm1_body,
        grid=(N // bp,),
        in_specs=[
            pl.BlockSpec((bp, D), lambda i: (i, 0)),
            pl.BlockSpec((bp, D), lambda i: (i, 0)),
            pl.BlockSpec((bp, 16), lambda i: (i, 0)),
            pl.BlockSpec((bp, 16), lambda i: (i, 0)),
            pl.BlockSpec((bp, ROW), lambda i: (i, 0)),
            pl.BlockSpec((1, D), lambda i: (0, 0)),
            pl.BlockSpec((16, D), lambda i: (0, 0)),
        ],
        out_specs=[
            pl.BlockSpec((bp, D), lambda i: (i, 0)),
            pl.BlockSpec((1, 2, D), lambda i: (i, 0, 0)),
        ],
        out_shape=[
            jax.ShapeDtypeStruct((N, D), jnp.float32),
            jax.ShapeDtypeStruct((N // bp, 2, D), jnp.float32),
        ],
    )(a0, a1, b0, b1, sf, bias1, R)


def _norm2_body(o_ref, st_ref, x_ref, g_ref, be_ref, out_ref):
    st = jnp.sum(st_ref[...], axis=0)
    m = st[0:1, :] * (1.0 / N)
    v = st[1:2, :] * (1.0 / N) - m * m
    inv = lax.rsqrt(v + 1e-5)
    out_ref[...] = (o_ref[...] - m) * inv * g_ref[...] + be_ref[...] + x_ref[...]


def _norm2_call(o, st, x, g1, b1):
    bp = 2000
    nb = N // bp
    return pl.pallas_call(
        _norm2_body,
        grid=(nb,),
        in_specs=[
            pl.BlockSpec((bp, D), lambda i: (i, 0)),
            pl.BlockSpec((nb, 2, D), lambda i: (0, 0, 0)),
            pl.BlockSpec((bp, D), lambda i: (i, 0)),
            pl.BlockSpec((1, D), lambda i: (0, 0)),
            pl.BlockSpec((1, D), lambda i: (0, 0)),
        ],
        out_specs=pl.BlockSpec((bp, D), lambda i: (i, 0)),
        out_shape=jax.ShapeDtypeStruct((N, D), jnp.float32),
    )(o, st, x, g1, b1)


def kernel(x, edge_index, W, att_src, att_dst, bias, gamma, beta):
    f32 = jnp.float32
    xp = jnp.zeros((N_TAB, D), f32).at[:N].set(x)
    oh16 = (jnp.arange(D)[:, None] // C == jnp.arange(16)[None, :]).astype(f32)
    A2 = jnp.concatenate([att_src.reshape(D)[:, None] * oh16,
                          att_dst.reshape(D)[:, None] * oh16], axis=1)
    R = oh16.T
    hs, adst, selfacc = _prep_call(xp, W, A2, R)

    a0, b0, a1, b1 = _edge_call(edge_index.reshape(2 * E), hs, adst)

    o, st = _norm1_call(a0, a1, b0, b1, selfacc, bias.reshape(1, D), R)
    return _norm2_call(o, st, x, gamma.reshape(1, D), beta.reshape(1, D))
